# Initial kernel scaffold; baseline (speedup 1.0000x reference)
import jax
import jax.numpy as jnp
from jax.experimental import pallas as pl


def kernel(edge_attr, emb, edge_index, batch_vec):
    raise NotImplementedError("write your pallas kernel here")


# speedup vs baseline: 3.0032x; 3.0032x over previous
import jax
import jax.numpy as jnp
from jax import lax
from jax.experimental import pallas as pl
from jax.experimental.pallas import tpu as pltpu

_B = 16
_M = 128
_N = 1024
_E = 32768
_D = 64


def _graph_kernel(pos_s, eid_s, starts_s, attr_ref, emb_ref, batch_ref, out_ref):
    b = pl.program_id(0)
    bv = batch_ref[...]
    cnt = jnp.minimum(jnp.sum(jnp.where(bv == b, 1, 0)), _M)

    r = lax.broadcasted_iota(jnp.int32, (_M * _M, _D), 0)
    i = r // _M
    j = r - i * _M
    isdiag = (i == j) & (i < cnt)
    e1 = emb_ref[1:2, :]
    e2 = emb_ref[2:3, :]
    out_ref[...] = jnp.where(isdiag, e1, e2)

    s0 = starts_s[b]
    s1 = starts_s[b + 1]

    def zero_body(k, carry):
        p = pos_s[k]
        out_ref[pl.ds(p, 1), :] = jnp.zeros((1, _D), jnp.float32)
        return carry

    lax.fori_loop(s0, s1, zero_body, 0)

    def add_body(k, carry):
        p = pos_s[k]
        e = eid_s[k]
        row = attr_ref[pl.ds(e, 1), :]
        out_ref[pl.ds(p, 1), :] += row
        return carry

    lax.fori_loop(s0, s1, add_body, 0)


def kernel(edge_attr, emb, edge_index, batch_vec):
    src = edge_index[0]
    dst = edge_index[1]
    counts = jnp.bincount(batch_vec, length=_B)
    cum = jnp.concatenate(
        [jnp.zeros((1,), counts.dtype), jnp.cumsum(counts)])[:-1]
    g = batch_vec[src]
    off = cum[g]
    i0 = src - off
    i1 = dst - off
    valid = (i0 >= 0) & (i0 < _M) & (i1 >= 0) & (i1 < _M)
    key = jnp.where(valid, g, _B).astype(jnp.int32)
    order = jnp.argsort(key).astype(jnp.int32)
    pos_s = (i0[order] * _M + i1[order]).astype(jnp.int32)
    eid_s = order
    starts = jnp.searchsorted(
        key[order], jnp.arange(_B + 1, dtype=jnp.int32)).astype(jnp.int32)

    emb_pad = jnp.zeros((8, _D), jnp.float32).at[:3].set(emb)
    batch_2d = batch_vec.reshape(8, 128).astype(jnp.int32)

    out = pl.pallas_call(
        _graph_kernel,
        grid_spec=pltpu.PrefetchScalarGridSpec(
            num_scalar_prefetch=3,
            grid=(_B,),
            in_specs=[
                pl.BlockSpec((_E, _D), lambda b, *_: (0, 0)),
                pl.BlockSpec((8, _D), lambda b, *_: (0, 0)),
                pl.BlockSpec((8, 128), lambda b, *_: (0, 0)),
            ],
            out_specs=pl.BlockSpec((_M * _M, _D), lambda b, *_: (b, 0)),
        ),
        out_shape=jax.ShapeDtypeStruct((_B * _M * _M, _D), jnp.float32),
    )(pos_s, eid_s, starts, edge_attr, emb_pad, batch_2d)
    return out.reshape(_B, _M, _M, _D)


# speedup vs baseline: 3.3534x; 1.1166x over previous
import jax
import jax.numpy as jnp
from jax import lax
from jax.experimental import pallas as pl
from jax.experimental.pallas import tpu as pltpu

_B = 16
_M = 128
_N = 1024
_E = 32768
_D = 64


def _graph_kernel(pos_s, eid_s, starts_s, attr_ref, emb_ref, batch_ref,
                  out_ref, bg_ref):
    b = pl.program_id(0)

    @pl.when(b == 0)
    def _():
        r = lax.broadcasted_iota(jnp.int32, (_M * _M, _D), 0)
        i = r >> 7
        j = r & (_M - 1)
        e1 = emb_ref[1:2, :]
        e2 = emb_ref[2:3, :]
        bg_ref[...] = jnp.where(i == j, e1, e2)

    out_ref[...] = bg_ref[...]

    bv = batch_ref[...]
    cnt = jnp.minimum(jnp.sum(jnp.where(bv == b, 1, 0)), _M)
    e2row = emb_ref[2:3, :]

    def fix_body(i, carry):
        out_ref[pl.ds(i * (_M + 1), 1), :] = e2row
        return carry

    lax.fori_loop(cnt, _M, fix_body, 0)

    s0 = starts_s[b]
    s1 = starts_s[b + 1]

    def zero_body(k, carry):
        p = pos_s[k]
        out_ref[pl.ds(p, 1), :] = jnp.zeros((1, _D), jnp.float32)
        return carry

    lax.fori_loop(s0, s1, zero_body, 0)

    def add_body(k, carry):
        p = pos_s[k]
        e = eid_s[k]
        row = attr_ref[pl.ds(e, 1), :]
        out_ref[pl.ds(p, 1), :] += row
        return carry

    lax.fori_loop(s0, s1, add_body, 0)


def kernel(edge_attr, emb, edge_index, batch_vec):
    src = edge_index[0]
    dst = edge_index[1]
    counts = jnp.bincount(batch_vec, length=_B)
    cum = jnp.concatenate(
        [jnp.zeros((1,), counts.dtype), jnp.cumsum(counts)])[:-1]
    g = batch_vec[src]
    off = cum[g]
    i0 = src - off
    i1 = dst - off
    valid = (i0 >= 0) & (i0 < _M) & (i1 >= 0) & (i1 < _M)
    key = jnp.where(valid, g, _B).astype(jnp.int32)
    order = jnp.argsort(key).astype(jnp.int32)
    pos_s = (i0[order] * _M + i1[order]).astype(jnp.int32)
    eid_s = order
    starts = jnp.searchsorted(
        key[order], jnp.arange(_B + 1, dtype=jnp.int32)).astype(jnp.int32)

    emb_pad = jnp.zeros((8, _D), jnp.float32).at[:3].set(emb)
    batch_2d = batch_vec.reshape(8, 128).astype(jnp.int32)

    out = pl.pallas_call(
        _graph_kernel,
        grid_spec=pltpu.PrefetchScalarGridSpec(
            num_scalar_prefetch=3,
            grid=(_B,),
            in_specs=[
                pl.BlockSpec((_E, _D), lambda b, *_: (0, 0)),
                pl.BlockSpec((8, _D), lambda b, *_: (0, 0)),
                pl.BlockSpec((8, 128), lambda b, *_: (0, 0)),
            ],
            out_specs=pl.BlockSpec((_M * _M, _D), lambda b, *_: (b, 0)),
            scratch_shapes=[pltpu.VMEM((_M * _M, _D), jnp.float32)],
        ),
        out_shape=jax.ShapeDtypeStruct((_B * _M * _M, _D), jnp.float32),
    )(pos_s, eid_s, starts, edge_attr, emb_pad, batch_2d)
    return out.reshape(_B, _M, _M, _D)
